# Initial kernel scaffold; baseline (speedup 1.0000x reference)
import jax
import jax.numpy as jnp
from jax.experimental import pallas as pl


def kernel(x, hs, rule_embeddings, pa_Wq, pa_bq, pa_Wk, pa_bk, pa_Wv, pa_bv, sa_Wq, sa_Wk, gru_Wih, gru_Whh, gru_bih, gru_bhh, c_Wq, c_Wk, c_Wv):
    raise NotImplementedError("write your pallas kernel here")


# speedup vs baseline: 2.8398x; 2.8398x over previous
import math

import jax
import jax.numpy as jnp
from jax.experimental import pallas as pl

_HIGH = jax.lax.Precision.HIGHEST

_NH = 4
_R = 8
_HEADS = 4
_CK = 32
_VH = 16

_bf16 = jnp.bfloat16
_f32 = jnp.float32


def _dot(a, b):
    return jnp.dot(a.astype(_bf16), b.astype(_bf16),
                   preferred_element_type=_f32)


def _rnd(a):
    return a.astype(_bf16).astype(_f32)


def _fused_kernel(x_ref, hs_ref, remb_ref, pa_Wq_ref, pa_bq_ref, pa_Wk_ref,
                  pa_bk_ref, pa_Wv_ref, pa_bv_ref, sa_Wq_ref, sa_Wk_ref,
                  Wih_ref, Whh_ref, bih_ref, bhh_ref, c_Wq_ref, c_Wk_ref,
                  c_Wv_ref, out_ref):
    Bt = x_ref.shape[0]
    RT = Bt * _NH
    Hd = hs_ref.shape[1]
    KD = pa_Wq_ref.shape[1]
    VD = pa_Wv_ref.shape[1]

    xb = x_ref[...]
    hsb = hs_ref[...]

    q = _dot(hsb, pa_Wq_ref[...]) + pa_bq_ref[...]
    k0 = _dot(xb, pa_Wk_ref[...]) + pa_bk_ref[...]
    v0 = _dot(xb, pa_Wv_ref[...]) + pa_bv_ref[...]
    krep = jnp.broadcast_to(k0[:, None, :], (Bt, _NH, KD)).reshape(RT, KD)
    vrep = jnp.broadcast_to(v0[:, None, :], (Bt, _NH, VD)).reshape(RT, VD)
    pa_scale = 1.0 / math.sqrt(KD)
    qr = _rnd(q)
    l0 = jnp.sum(qr * _rnd(krep), axis=1, keepdims=True) * pa_scale
    l1 = jnp.sum(qr * _rnd(pa_bk_ref[...]), axis=1, keepdims=True) * pa_scale
    lm = jnp.maximum(l0, l1)
    a0 = jnp.exp(l0 - lm)
    a1 = jnp.exp(l1 - lm)
    inv = 1.0 / (a0 + a1)
    attn0 = a0 * inv
    attn1 = a1 * inv
    inputs_b = _rnd(attn0) * _rnd(vrep) + _rnd(attn1) * _rnd(pa_bv_ref[...])

    qs = _dot(hsb, sa_Wq_ref[:Hd, :]) + _dot(inputs_b, sa_Wq_ref[Hd:, :])
    ksel = _dot(remb_ref[...], sa_Wk_ref[...])
    scores = jax.lax.dot_general(
        qs.astype(_bf16), ksel.astype(_bf16), (((1,), (1,)), ((), ())),
        preferred_element_type=_f32) * (1.0 / math.sqrt(qs.shape[1]))
    smax = jnp.max(scores, axis=1, keepdims=True)
    lane = jax.lax.broadcasted_iota(jnp.int32, scores.shape, 1)
    cand = jnp.where(scores == smax, lane, _R)
    sel = jnp.min(cand, axis=1, keepdims=True)
    mask = (lane == sel).astype(_f32)

    px = jnp.dot(mask, bih_ref[...], precision=_HIGH,
                 preferred_element_type=_f32)
    ph = jnp.dot(mask, bhh_ref[...], precision=_HIGH,
                 preferred_element_type=_f32)
    for r in range(_R):
        mr = mask[:, r:r + 1]
        px = px + mr * _dot(inputs_b, Wih_ref[r])
        ph = ph + mr * _dot(hsb, Whh_ref[r])
    rg = jax.nn.sigmoid(px[:, 0:Hd] + ph[:, 0:Hd])
    zg = jax.nn.sigmoid(px[:, Hd:2 * Hd] + ph[:, Hd:2 * Hd])
    ng = jnp.tanh(px[:, 2 * Hd:3 * Hd] + rg * ph[:, 2 * Hd:3 * Hd])
    hnew = (1.0 - zg) * ng + zg * hsb

    qc = _dot(hnew, c_Wq_ref[...])
    kc = _dot(hnew, c_Wk_ref[...])
    vc = _dot(hnew, c_Wv_ref[...])
    q3 = qc.reshape(Bt, _NH, _HEADS * _CK)
    k3 = kc.reshape(Bt, _NH, _HEADS * _CK)
    v3 = vc.reshape(Bt, _NH, _HEADS * _VH)
    kidx = jax.lax.broadcasted_iota(jnp.int32, (_HEADS * _CK, _HEADS), 0) // _CK
    hidx = jax.lax.broadcasted_iota(jnp.int32, (_HEADS * _CK, _HEADS), 1)
    S = (kidx == hidx).astype(_f32)
    hidx2 = jax.lax.broadcasted_iota(jnp.int32, (_HEADS, _HEADS * _VH), 0)
    cidx = jax.lax.broadcasted_iota(jnp.int32, (_HEADS, _HEADS * _VH), 1) // _VH
    E = (hidx2 == cidx).astype(_f32)
    c_scale = 1.0 / math.sqrt(_CK)
    row_slot = jax.lax.rem(
        jax.lax.broadcasted_iota(jnp.int32, (RT, 1), 0), _NH)
    ctx_full = jnp.zeros((RT, Hd), _f32)
    for n in range(_NH):
        Qn = _rnd(q3[:, n, :])
        ls = [_dot(Qn * _rnd(k3[:, mi, :]), S) * c_scale
              for mi in range(_NH)]
        lmax = jnp.maximum(jnp.maximum(ls[0], ls[1]),
                           jnp.maximum(ls[2], ls[3]))
        es = [jnp.exp(l - lmax) for l in ls]
        invden = 1.0 / (es[0] + es[1] + es[2] + es[3])
        ctx_n = jnp.zeros((Bt, Hd), _f32)
        for mi in range(_NH):
            p = es[mi] * invden
            ctx_n = ctx_n + _dot(p, E) * v3[:, mi, :]
        ctx_rep = jnp.broadcast_to(ctx_n[:, None, :], (Bt, _NH, Hd)).reshape(RT, Hd)
        ctx_full = ctx_full + jnp.where(row_slot == n, 1.0, 0.0) * ctx_rep
    out_ref[...] = hnew + ctx_full


def kernel(x, hs, rule_embeddings, pa_Wq, pa_bq, pa_Wk, pa_bk, pa_Wv, pa_bv,
           sa_Wq, sa_Wk, gru_Wih, gru_Whh, gru_bih, gru_bhh, c_Wq, c_Wk, c_Wv):
    B, IN = x.shape
    NH, Hd = hs.shape[1], hs.shape[2]
    Bt = 256
    G = B // Bt
    hs_f = hs.reshape(B * NH, Hd)
    Wih_t = gru_Wih.transpose(0, 2, 1)
    Whh_t = gru_Whh.transpose(0, 2, 1)

    def row2(v):
        return v.reshape(1, -1)

    full2 = lambda a: pl.BlockSpec(a.shape, lambda i: (0, 0))
    full3 = lambda a: pl.BlockSpec(a.shape, lambda i: (0, 0, 0))
    out = pl.pallas_call(
        _fused_kernel,
        grid=(G,),
        in_specs=[
            pl.BlockSpec((Bt, IN), lambda i: (i, 0)),
            pl.BlockSpec((Bt * NH, Hd), lambda i: (i, 0)),
            full2(rule_embeddings),
            full2(pa_Wq), full2(row2(pa_bq)),
            full2(pa_Wk), full2(row2(pa_bk)),
            full2(pa_Wv), full2(row2(pa_bv)),
            full2(sa_Wq), full2(sa_Wk),
            full3(Wih_t), full3(Whh_t),
            full2(gru_bih), full2(gru_bhh),
            full2(c_Wq), full2(c_Wk), full2(c_Wv),
        ],
        out_specs=pl.BlockSpec((Bt * NH, Hd), lambda i: (i, 0)),
        out_shape=jax.ShapeDtypeStruct((B * NH, Hd), jnp.float32),
    )(x, hs_f, rule_embeddings, pa_Wq, row2(pa_bq), pa_Wk, row2(pa_bk),
      pa_Wv, row2(pa_bv), sa_Wq, sa_Wk, Wih_t, Whh_t, gru_bih, gru_bhh,
      c_Wq, c_Wk, c_Wv)
    return out.reshape(B, NH, Hd)


# speedup vs baseline: 3.1911x; 1.1237x over previous
import math

import jax
import jax.numpy as jnp
from jax.experimental import pallas as pl

_NH = 4
_R = 8
_HEADS = 4
_CK = 32
_VH = 16

_bf16 = jnp.bfloat16
_f32 = jnp.float32


def _dot(a, b):
    return jnp.dot(a, b, preferred_element_type=_f32)


def _rnd(a):
    return a.astype(_bf16).astype(_f32)


def _fused_kernel(x_ref, hs_ref, remb_ref, pa_Wq_ref, pa_bq_ref, pa_Wk_ref,
                  pa_bk_ref, pa_Wv_ref, pa_bv_ref, sa_Wq1_ref, sa_Wq2_ref,
                  sa_Wk_ref, Wih_ref, Whh_ref, bih_ref, bhh_ref, cq_ref,
                  ck_ref, cv_ref, out_ref):
    Bt = x_ref.shape[0]
    RT = Bt * _NH
    Hd = hs_ref.shape[1]
    KD = pa_Wq_ref.shape[1]
    VD = pa_Wv_ref.shape[1]

    xb = x_ref[...]
    hsb = hs_ref[...]
    xb_bf = xb.astype(_bf16)
    hs_bf = hsb.astype(_bf16)

    q = _dot(hs_bf, pa_Wq_ref[...]) + pa_bq_ref[...]
    k0 = _dot(xb_bf, pa_Wk_ref[...]) + pa_bk_ref[...]
    v0 = _dot(xb_bf, pa_Wv_ref[...]) + pa_bv_ref[...]
    k0r = _rnd(k0)
    v0r = _rnd(v0)
    krep = jnp.broadcast_to(k0r[:, None, :], (Bt, _NH, KD)).reshape(RT, KD)
    vrep = jnp.broadcast_to(v0r[:, None, :], (Bt, _NH, VD)).reshape(RT, VD)
    pa_scale = 1.0 / math.sqrt(KD)
    qr = _rnd(q)
    l0 = jnp.sum(qr * krep, axis=1, keepdims=True) * pa_scale
    l1 = jnp.sum(qr * _rnd(pa_bk_ref[...]), axis=1, keepdims=True) * pa_scale
    lm = jnp.maximum(l0, l1)
    a0 = jnp.exp(l0 - lm)
    a1 = jnp.exp(l1 - lm)
    inv = 1.0 / (a0 + a1)
    attn0 = a0 * inv
    attn1 = a1 * inv
    inputs_b = _rnd(attn0) * vrep + _rnd(attn1) * _rnd(pa_bv_ref[...])
    inputs_bf = inputs_b.astype(_bf16)

    qs = _dot(hs_bf, sa_Wq1_ref[...]) + _dot(inputs_bf, sa_Wq2_ref[...])
    ksel = _dot(remb_ref[...].astype(_bf16), sa_Wk_ref[...])
    scores = jax.lax.dot_general(
        qs.astype(_bf16), ksel.astype(_bf16), (((1,), (1,)), ((), ())),
        preferred_element_type=_f32) * (1.0 / math.sqrt(qs.shape[1]))
    smax = jnp.max(scores, axis=1, keepdims=True)
    lane = jax.lax.broadcasted_iota(jnp.int32, scores.shape, 1)
    cand = jnp.where(scores == smax, lane, _R)
    sel = jnp.min(cand, axis=1, keepdims=True)
    mask_bf = (lane == sel).astype(_bf16)

    px = _dot(mask_bf, bih_ref[...])
    ph = _dot(mask_bf, bhh_ref[...])
    for r in range(_R):
        mr = mask_bf[:, r:r + 1]
        px = px + _dot(mr * inputs_bf, Wih_ref[r])
        ph = ph + _dot(mr * hs_bf, Whh_ref[r])
    rg = jax.nn.sigmoid(px[:, 0:Hd] + ph[:, 0:Hd])
    zg = jax.nn.sigmoid(px[:, Hd:2 * Hd] + ph[:, Hd:2 * Hd])
    ng = jnp.tanh(px[:, 2 * Hd:3 * Hd] + rg * ph[:, 2 * Hd:3 * Hd])
    hnew = (1.0 - zg) * ng + zg * hsb

    h3 = hnew.reshape(Bt, _NH, Hd)
    hcat = jnp.concatenate([h3[:, n, :] for n in range(_NH)], axis=1)
    hcat_bf = hcat.astype(_bf16)
    QK = _HEADS * _CK
    qcat = _dot(hcat_bf, cq_ref[...])
    kcat = _dot(hcat_bf, ck_ref[...])
    vcat = _dot(hcat_bf, cv_ref[...])
    kcat_bf = kcat.astype(_bf16)
    lidx = jax.lax.broadcasted_iota(jnp.int32, (_NH * QK, _NH * _HEADS), 0)
    jidx = jax.lax.broadcasted_iota(jnp.int32, (_NH * QK, _NH * _HEADS), 1)
    S2 = jnp.where((lidx // QK == jidx // _HEADS)
                   & ((lidx % QK) // _CK == jidx % _HEADS), 1.0, 0.0).astype(_bf16)
    hidx2 = jax.lax.broadcasted_iota(jnp.int32, (_HEADS, _HEADS * _VH), 0)
    cidx = jax.lax.broadcasted_iota(jnp.int32, (_HEADS, _HEADS * _VH), 1) // _VH
    E = (hidx2 == cidx).astype(_bf16)
    c_scale = 1.0 / math.sqrt(_CK)
    ctx = []
    for n in range(_NH):
        qn = qcat[:, n * QK:(n + 1) * QK].astype(_bf16)
        qn_rep = jnp.concatenate([qn] * _NH, axis=1)
        ls_all = _dot(qn_rep * kcat_bf, S2) * c_scale
        ls = [ls_all[:, mi * _HEADS:(mi + 1) * _HEADS] for mi in range(_NH)]
        lmax = jnp.maximum(jnp.maximum(ls[0], ls[1]),
                           jnp.maximum(ls[2], ls[3]))
        es = [jnp.exp(l - lmax) for l in ls]
        invden = 1.0 / (es[0] + es[1] + es[2] + es[3])
        ctx_n = jnp.zeros((Bt, Hd), _f32)
        for mi in range(_NH):
            p = (es[mi] * invden).astype(_bf16)
            ctx_n = ctx_n + _dot(p, E) * vcat[:, mi * Hd:(mi + 1) * Hd]
        ctx.append(ctx_n)
    out_ref[...] = hcat + jnp.concatenate(ctx, axis=1)


def kernel(x, hs, rule_embeddings, pa_Wq, pa_bq, pa_Wk, pa_bk, pa_Wv, pa_bv,
           sa_Wq, sa_Wk, gru_Wih, gru_Whh, gru_bih, gru_bhh, c_Wq, c_Wk, c_Wv):
    B, IN = x.shape
    NH, Hd = hs.shape[1], hs.shape[2]
    Bt = 256
    G = B // Bt
    hs_f = hs.reshape(B * NH, Hd)
    bf = lambda a: a.astype(_bf16)
    Wih_t = bf(gru_Wih.transpose(0, 2, 1))
    Whh_t = bf(gru_Whh.transpose(0, 2, 1))
    eye = jnp.eye(NH, dtype=jnp.float32)
    cq_blk = bf(jnp.kron(eye, c_Wq))
    ck_blk = bf(jnp.kron(eye, c_Wk))
    cv_blk = bf(jnp.kron(eye, c_Wv))

    def row2(v):
        return v.reshape(1, -1)

    full2 = lambda a: pl.BlockSpec(a.shape, lambda i: (0, 0))
    full3 = lambda a: pl.BlockSpec(a.shape, lambda i: (0, 0, 0))
    args = (x, hs_f, rule_embeddings, bf(pa_Wq), row2(pa_bq), bf(pa_Wk),
            row2(pa_bk), bf(pa_Wv), row2(pa_bv), bf(sa_Wq[:Hd]),
            bf(sa_Wq[Hd:]), bf(sa_Wk), Wih_t, Whh_t, bf(gru_bih),
            bf(gru_bhh), cq_blk, ck_blk, cv_blk)
    out = pl.pallas_call(
        _fused_kernel,
        grid=(G,),
        in_specs=[
            pl.BlockSpec((Bt, IN), lambda i: (i, 0)),
            pl.BlockSpec((Bt * NH, Hd), lambda i: (i, 0)),
        ] + [full3(a) if a.ndim == 3 else full2(a) for a in args[2:]],
        out_specs=pl.BlockSpec((Bt, NH * Hd), lambda i: (i, 0)),
        out_shape=jax.ShapeDtypeStruct((B, NH * Hd), jnp.float32),
    )(*args)
    return out.reshape(B, NH, Hd)


# speedup vs baseline: 3.7103x; 1.1627x over previous
import math

import jax
import jax.numpy as jnp
from jax.experimental import pallas as pl

_NH = 4
_R = 8
_HEADS = 4
_CK = 32
_VH = 16

_bf16 = jnp.bfloat16
_f32 = jnp.float32


def _dot(a, b):
    return jnp.dot(a, b, preferred_element_type=_f32)


def _rnd(a):
    return a.astype(_bf16).astype(_f32)


def _fused_kernel(x_ref, hs_ref, remb_ref, pa_Wq_ref, pa_bq_ref, pa_Wk_ref,
                  pa_bk_ref, pa_Wv_ref, pa_bv_ref, sa_Wq1_ref, sa_Wq2_ref,
                  sa_Wk_ref, Wih_ref, Whh_ref, bih_ref, bhh_ref, cq_ref,
                  ck_ref, cv_ref, out_ref):
    Bt = x_ref.shape[0]
    RT = Bt * _NH
    Hd = hs_ref.shape[1]
    KD = pa_Wq_ref.shape[1]
    VD = pa_Wv_ref.shape[1]

    xb = x_ref[...]
    hsb = hs_ref[...]
    xb_bf = xb.astype(_bf16)
    hs_bf = hsb.astype(_bf16)

    q = _dot(hs_bf, pa_Wq_ref[...]) + pa_bq_ref[...]
    k0 = _dot(xb_bf, pa_Wk_ref[...]) + pa_bk_ref[...]
    v0 = _dot(xb_bf, pa_Wv_ref[...]) + pa_bv_ref[...]
    k0r = _rnd(k0)
    v0r = _rnd(v0)
    krep = jnp.broadcast_to(k0r[:, None, :], (Bt, _NH, KD)).reshape(RT, KD)
    vrep = jnp.broadcast_to(v0r[:, None, :], (Bt, _NH, VD)).reshape(RT, VD)
    pa_scale = 1.0 / math.sqrt(KD)
    qr = _rnd(q)
    l0 = jnp.sum(qr * krep, axis=1, keepdims=True) * pa_scale
    l1 = jnp.sum(qr * _rnd(pa_bk_ref[...]), axis=1, keepdims=True) * pa_scale
    lm = jnp.maximum(l0, l1)
    a0 = jnp.exp(l0 - lm)
    a1 = jnp.exp(l1 - lm)
    inv = 1.0 / (a0 + a1)
    attn0 = a0 * inv
    attn1 = a1 * inv
    inputs_b = _rnd(attn0) * vrep + _rnd(attn1) * _rnd(pa_bv_ref[...])
    inputs_bf = inputs_b.astype(_bf16)

    qs = _dot(hs_bf, sa_Wq1_ref[...]) + _dot(inputs_bf, sa_Wq2_ref[...])
    ksel = _dot(remb_ref[...].astype(_bf16), sa_Wk_ref[...])
    scores = jax.lax.dot_general(
        qs.astype(_bf16), ksel.astype(_bf16), (((1,), (1,)), ((), ())),
        preferred_element_type=_f32) * (1.0 / math.sqrt(qs.shape[1]))
    smax = jnp.max(scores, axis=1, keepdims=True)
    lane = jax.lax.broadcasted_iota(jnp.int32, scores.shape, 1)
    cand = jnp.where(scores == smax, lane, _R)
    sel = jnp.min(cand, axis=1, keepdims=True)
    mask_bf = (lane == sel).astype(_bf16)

    px = _dot(mask_bf, bih_ref[...])
    ph = _dot(mask_bf, bhh_ref[...])
    for r in range(_R):
        mr = mask_bf[:, r:r + 1]
        px = px + _dot(mr * inputs_bf, Wih_ref[r])
        ph = ph + _dot(mr * hs_bf, Whh_ref[r])
    rg = jax.nn.sigmoid(px[:, 0:Hd] + ph[:, 0:Hd])
    zg = jax.nn.sigmoid(px[:, Hd:2 * Hd] + ph[:, Hd:2 * Hd])
    ng = jnp.tanh(px[:, 2 * Hd:3 * Hd] + rg * ph[:, 2 * Hd:3 * Hd])
    hnew = (1.0 - zg) * ng + zg * hsb

    h3 = hnew.reshape(Bt, _NH, Hd)
    hcat = jnp.concatenate([h3[:, n, :] for n in range(_NH)], axis=1)
    hcat_bf = hcat.astype(_bf16)
    QK = _HEADS * _CK
    qcat = _dot(hcat_bf, cq_ref[...])
    kcat = _dot(hcat_bf, ck_ref[...])
    vcat = _dot(hcat_bf, cv_ref[...])
    kcat_bf = kcat.astype(_bf16)
    lidx = jax.lax.broadcasted_iota(jnp.int32, (_NH * QK, _NH * _HEADS), 0)
    jidx = jax.lax.broadcasted_iota(jnp.int32, (_NH * QK, _NH * _HEADS), 1)
    S2 = jnp.where((lidx // QK == jidx // _HEADS)
                   & ((lidx % QK) // _CK == jidx % _HEADS), 1.0, 0.0).astype(_bf16)
    hidx2 = jax.lax.broadcasted_iota(jnp.int32, (_HEADS, _HEADS * _VH), 0)
    cidx = jax.lax.broadcasted_iota(jnp.int32, (_HEADS, _HEADS * _VH), 1) // _VH
    E = (hidx2 == cidx).astype(_bf16)
    c_scale = 1.0 / math.sqrt(_CK)
    ctx = []
    for n in range(_NH):
        qn = qcat[:, n * QK:(n + 1) * QK].astype(_bf16)
        qn_rep = jnp.concatenate([qn] * _NH, axis=1)
        ls_all = _dot(qn_rep * kcat_bf, S2) * c_scale
        ls = [ls_all[:, mi * _HEADS:(mi + 1) * _HEADS] for mi in range(_NH)]
        lmax = jnp.maximum(jnp.maximum(ls[0], ls[1]),
                           jnp.maximum(ls[2], ls[3]))
        es = [jnp.exp(l - lmax) for l in ls]
        invden = 1.0 / (es[0] + es[1] + es[2] + es[3])
        ctx_n = jnp.zeros((Bt, Hd), _f32)
        for mi in range(_NH):
            p = (es[mi] * invden).astype(_bf16)
            ctx_n = ctx_n + _dot(p, E) * vcat[:, mi * Hd:(mi + 1) * Hd]
        ctx.append(ctx_n)
    out_ref[...] = hcat + jnp.concatenate(ctx, axis=1)


def kernel(x, hs, rule_embeddings, pa_Wq, pa_bq, pa_Wk, pa_bk, pa_Wv, pa_bv,
           sa_Wq, sa_Wk, gru_Wih, gru_Whh, gru_bih, gru_bhh, c_Wq, c_Wk, c_Wv):
    B, IN = x.shape
    NH, Hd = hs.shape[1], hs.shape[2]
    Bt = 512
    G = B // Bt
    hs_f = hs.reshape(B * NH, Hd)
    bf = lambda a: a.astype(_bf16)
    Wih_t = bf(gru_Wih.transpose(0, 2, 1))
    Whh_t = bf(gru_Whh.transpose(0, 2, 1))
    eye = jnp.eye(NH, dtype=jnp.float32)
    cq_blk = bf(jnp.kron(eye, c_Wq))
    ck_blk = bf(jnp.kron(eye, c_Wk))
    cv_blk = bf(jnp.kron(eye, c_Wv))

    def row2(v):
        return v.reshape(1, -1)

    full2 = lambda a: pl.BlockSpec(a.shape, lambda i: (0, 0))
    full3 = lambda a: pl.BlockSpec(a.shape, lambda i: (0, 0, 0))
    args = (x, hs_f, rule_embeddings, bf(pa_Wq), row2(pa_bq), bf(pa_Wk),
            row2(pa_bk), bf(pa_Wv), row2(pa_bv), bf(sa_Wq[:Hd]),
            bf(sa_Wq[Hd:]), bf(sa_Wk), Wih_t, Whh_t, bf(gru_bih),
            bf(gru_bhh), cq_blk, ck_blk, cv_blk)
    out = pl.pallas_call(
        _fused_kernel,
        grid=(G,),
        in_specs=[
            pl.BlockSpec((Bt, IN), lambda i: (i, 0)),
            pl.BlockSpec((Bt * NH, Hd), lambda i: (i, 0)),
        ] + [full3(a) if a.ndim == 3 else full2(a) for a in args[2:]],
        out_specs=pl.BlockSpec((Bt, NH * Hd), lambda i: (i, 0)),
        out_shape=jax.ShapeDtypeStruct((B, NH * Hd), jnp.float32),
    )(*args)
    return out.reshape(B, NH, Hd)


# speedup vs baseline: 3.8375x; 1.0343x over previous
import math

import jax
import jax.numpy as jnp
from jax.experimental import pallas as pl

_NH = 4
_R = 8
_HEADS = 4
_CK = 32
_VH = 16

_bf16 = jnp.bfloat16
_f32 = jnp.float32


def _dot(a, b):
    return jnp.dot(a, b, preferred_element_type=_f32)


def _rnd(a):
    return a.astype(_bf16).astype(_f32)


def _fused_kernel(x_ref, hs_ref, remb_ref, pa_Wq_ref, pa_bq_ref, pa_Wk_ref,
                  pa_bk_ref, pa_Wv_ref, pa_bv_ref, sa_Wq1_ref, sa_Wq2_ref,
                  sa_Wk_ref, Wih_ref, Whh_ref, bih_ref, bhh_ref, cq_ref,
                  ck_ref, cv_ref, out_ref):
    Bt = x_ref.shape[0]
    RT = Bt * _NH
    Hd = hs_ref.shape[1]
    KD = pa_Wq_ref.shape[1]
    VD = pa_Wv_ref.shape[1]

    xb = x_ref[...]
    hsb = hs_ref[...]
    xb_bf = xb.astype(_bf16)
    hs_bf = hsb.astype(_bf16)

    q = _dot(hs_bf, pa_Wq_ref[...]) + pa_bq_ref[...]
    k0 = _dot(xb_bf, pa_Wk_ref[...]) + pa_bk_ref[...]
    v0 = _dot(xb_bf, pa_Wv_ref[...]) + pa_bv_ref[...]
    k0r = _rnd(k0)
    v0r = _rnd(v0)
    krep = jnp.broadcast_to(k0r[:, None, :], (Bt, _NH, KD)).reshape(RT, KD)
    vrep = jnp.broadcast_to(v0r[:, None, :], (Bt, _NH, VD)).reshape(RT, VD)
    pa_scale = 1.0 / math.sqrt(KD)
    qr = _rnd(q)
    l0 = jnp.sum(qr * krep, axis=1, keepdims=True) * pa_scale
    l1 = jnp.sum(qr * _rnd(pa_bk_ref[...]), axis=1, keepdims=True) * pa_scale
    lm = jnp.maximum(l0, l1)
    a0 = jnp.exp(l0 - lm)
    a1 = jnp.exp(l1 - lm)
    inv = 1.0 / (a0 + a1)
    attn0 = a0 * inv
    attn1 = a1 * inv
    inputs_b = _rnd(attn0) * vrep + _rnd(attn1) * _rnd(pa_bv_ref[...])
    inputs_bf = inputs_b.astype(_bf16)

    qs = _dot(hs_bf, sa_Wq1_ref[...]) + _dot(inputs_bf, sa_Wq2_ref[...])
    ksel = _dot(remb_ref[...].astype(_bf16), sa_Wk_ref[...])
    scores = jax.lax.dot_general(
        qs.astype(_bf16), ksel.astype(_bf16), (((1,), (1,)), ((), ())),
        preferred_element_type=_f32) * (1.0 / math.sqrt(qs.shape[1]))
    smax = jnp.max(scores, axis=1, keepdims=True)
    lane = jax.lax.broadcasted_iota(jnp.int32, scores.shape, 1)
    cand = jnp.where(scores == smax, lane, _R)
    sel = jnp.min(cand, axis=1, keepdims=True)
    mask_bf = (lane == sel).astype(_bf16)

    px = _dot(mask_bf, bih_ref[...])
    ph = _dot(mask_bf, bhh_ref[...])
    for r in range(_R):
        mr = mask_bf[:, r:r + 1]
        px = px + _dot(mr * inputs_bf, Wih_ref[r])
        ph = ph + _dot(mr * hs_bf, Whh_ref[r])
    rg = jax.nn.sigmoid(px[:, 0:Hd] + ph[:, 0:Hd])
    zg = jax.nn.sigmoid(px[:, Hd:2 * Hd] + ph[:, Hd:2 * Hd])
    ng = jnp.tanh(px[:, 2 * Hd:3 * Hd] + rg * ph[:, 2 * Hd:3 * Hd])
    hnew = (1.0 - zg) * ng + zg * hsb

    h3 = hnew.reshape(Bt, _NH, Hd)
    hcat = jnp.concatenate([h3[:, n, :] for n in range(_NH)], axis=1)
    hcat_bf = hcat.astype(_bf16)
    QK = _HEADS * _CK
    qcat = _dot(hcat_bf, cq_ref[...])
    kcat = _dot(hcat_bf, ck_ref[...])
    vcat = _dot(hcat_bf, cv_ref[...])
    kcat_bf = kcat.astype(_bf16)
    lidx = jax.lax.broadcasted_iota(jnp.int32, (_NH * QK, _NH * _HEADS), 0)
    jidx = jax.lax.broadcasted_iota(jnp.int32, (_NH * QK, _NH * _HEADS), 1)
    S2 = jnp.where((lidx // QK == jidx // _HEADS)
                   & ((lidx % QK) // _CK == jidx % _HEADS), 1.0, 0.0).astype(_bf16)
    hidx2 = jax.lax.broadcasted_iota(jnp.int32, (_HEADS, _HEADS * _VH), 0)
    cidx = jax.lax.broadcasted_iota(jnp.int32, (_HEADS, _HEADS * _VH), 1) // _VH
    E = (hidx2 == cidx).astype(_bf16)
    c_scale = 1.0 / math.sqrt(_CK)
    ctx = []
    for n in range(_NH):
        qn = qcat[:, n * QK:(n + 1) * QK].astype(_bf16)
        qn_rep = jnp.concatenate([qn] * _NH, axis=1)
        ls_all = _dot(qn_rep * kcat_bf, S2) * c_scale
        ls = [ls_all[:, mi * _HEADS:(mi + 1) * _HEADS] for mi in range(_NH)]
        lmax = jnp.maximum(jnp.maximum(ls[0], ls[1]),
                           jnp.maximum(ls[2], ls[3]))
        es = [jnp.exp(l - lmax) for l in ls]
        invden = 1.0 / (es[0] + es[1] + es[2] + es[3])
        ctx_n = jnp.zeros((Bt, Hd), _f32)
        for mi in range(_NH):
            p = (es[mi] * invden).astype(_bf16)
            ctx_n = ctx_n + _dot(p, E) * vcat[:, mi * Hd:(mi + 1) * Hd]
        ctx.append(ctx_n)
    out_ref[...] = hcat + jnp.concatenate(ctx, axis=1)


def kernel(x, hs, rule_embeddings, pa_Wq, pa_bq, pa_Wk, pa_bk, pa_Wv, pa_bv,
           sa_Wq, sa_Wk, gru_Wih, gru_Whh, gru_bih, gru_bhh, c_Wq, c_Wk, c_Wv):
    B, IN = x.shape
    NH, Hd = hs.shape[1], hs.shape[2]
    Bt = 1024
    G = B // Bt
    hs_f = hs.reshape(B * NH, Hd)
    bf = lambda a: a.astype(_bf16)
    Wih_t = bf(gru_Wih.transpose(0, 2, 1))
    Whh_t = bf(gru_Whh.transpose(0, 2, 1))
    eye = jnp.eye(NH, dtype=jnp.float32)
    cq_blk = bf(jnp.kron(eye, c_Wq))
    ck_blk = bf(jnp.kron(eye, c_Wk))
    cv_blk = bf(jnp.kron(eye, c_Wv))

    def row2(v):
        return v.reshape(1, -1)

    full2 = lambda a: pl.BlockSpec(a.shape, lambda i: (0, 0))
    full3 = lambda a: pl.BlockSpec(a.shape, lambda i: (0, 0, 0))
    args = (x, hs_f, rule_embeddings, bf(pa_Wq), row2(pa_bq), bf(pa_Wk),
            row2(pa_bk), bf(pa_Wv), row2(pa_bv), bf(sa_Wq[:Hd]),
            bf(sa_Wq[Hd:]), bf(sa_Wk), Wih_t, Whh_t, bf(gru_bih),
            bf(gru_bhh), cq_blk, ck_blk, cv_blk)
    out = pl.pallas_call(
        _fused_kernel,
        grid=(G,),
        in_specs=[
            pl.BlockSpec((Bt, IN), lambda i: (i, 0)),
            pl.BlockSpec((Bt * NH, Hd), lambda i: (i, 0)),
        ] + [full3(a) if a.ndim == 3 else full2(a) for a in args[2:]],
        out_specs=pl.BlockSpec((Bt, NH * Hd), lambda i: (i, 0)),
        out_shape=jax.ShapeDtypeStruct((B, NH * Hd), jnp.float32),
    )(*args)
    return out.reshape(B, NH, Hd)


# speedup vs baseline: 4.4559x; 1.1612x over previous
import math

import jax
import jax.numpy as jnp
from jax.experimental import pallas as pl

_NH = 4
_R = 8
_HEADS = 4
_CK = 32
_VH = 16

_bf16 = jnp.bfloat16
_f32 = jnp.float32


def _dot(a, b):
    return jnp.dot(a, b, preferred_element_type=_f32)


def _rnd(a):
    return a.astype(_bf16).astype(_f32)


def _fused_kernel(x_ref, hs_ref, remb_ref, pa_Wq_ref, pa_bq_ref, pa_Wk_ref,
                  pa_bk_ref, pa_Wv_ref, pa_bv_ref, sa_Wq_ref, sa_Wk_ref,
                  Wih_ref, Whh_ref, bih_ref, bhh_ref, cq_ref, ck_ref, cv_ref,
                  S3_ref, Gden_ref, Expand_ref, Collapse_ref, out_ref):
    Bt = x_ref.shape[0]
    RT = Bt * _NH
    Hd = hs_ref.shape[1]
    KD = pa_Wq_ref.shape[1]
    VD = pa_Wv_ref.shape[1]

    xb_bf = x_ref[...].astype(_bf16)
    hsb = hs_ref[...]
    hs_bf = hsb.astype(_bf16)

    q = _dot(hs_bf, pa_Wq_ref[...]) + pa_bq_ref[...]
    k0 = _dot(xb_bf, pa_Wk_ref[...]) + pa_bk_ref[...]
    v0 = _dot(xb_bf, pa_Wv_ref[...]) + pa_bv_ref[...]
    k0r = _rnd(k0)
    v0r = _rnd(v0)
    krep = jnp.broadcast_to(k0r[:, None, :], (Bt, _NH, KD)).reshape(RT, KD)
    vrep = jnp.broadcast_to(v0r[:, None, :], (Bt, _NH, VD)).reshape(RT, VD)
    pa_scale = 1.0 / math.sqrt(KD)
    qr = _rnd(q)
    l0 = jnp.sum(qr * krep, axis=1, keepdims=True) * pa_scale
    l1 = jnp.sum(qr * _rnd(pa_bk_ref[...]), axis=1, keepdims=True) * pa_scale
    lm = jnp.maximum(l0, l1)
    a0 = jnp.exp(l0 - lm)
    a1 = jnp.exp(l1 - lm)
    inv = 1.0 / (a0 + a1)
    attn0 = a0 * inv
    attn1 = a1 * inv
    inputs_b = _rnd(attn0) * vrep + _rnd(attn1) * _rnd(pa_bv_ref[...])
    inputs_bf = inputs_b.astype(_bf16)

    cat_ih = jnp.concatenate([hs_bf, inputs_bf], axis=1)
    qs = _dot(cat_ih, sa_Wq_ref[...])
    ksel = _dot(remb_ref[...].astype(_bf16), sa_Wk_ref[...])
    scores = jax.lax.dot_general(
        qs.astype(_bf16), ksel.astype(_bf16), (((1,), (1,)), ((), ())),
        preferred_element_type=_f32) * (1.0 / math.sqrt(qs.shape[1]))
    smax = jnp.max(scores, axis=1, keepdims=True)
    lane = jax.lax.broadcasted_iota(jnp.int32, scores.shape, 1)
    cand = jnp.where(scores == smax, lane, _R)
    sel = jnp.min(cand, axis=1, keepdims=True)
    mask_bf = (lane == sel).astype(_bf16)

    xbig = jnp.concatenate(
        [mask_bf[:, r:r + 1] * inputs_bf for r in range(_R)], axis=1)
    hbig = jnp.concatenate(
        [mask_bf[:, r:r + 1] * hs_bf for r in range(_R)], axis=1)
    px = _dot(xbig, Wih_ref[...]) + _dot(mask_bf, bih_ref[...])
    ph = _dot(hbig, Whh_ref[...]) + _dot(mask_bf, bhh_ref[...])
    rg = jax.nn.sigmoid(px[:, 0:Hd] + ph[:, 0:Hd])
    zg = jax.nn.sigmoid(px[:, Hd:2 * Hd] + ph[:, Hd:2 * Hd])
    ng = jnp.tanh(px[:, 2 * Hd:3 * Hd] + rg * ph[:, 2 * Hd:3 * Hd])
    hnew = (1.0 - zg) * ng + zg * hsb

    h3 = hnew.reshape(Bt, _NH, Hd)
    hcat = jnp.concatenate([h3[:, n, :] for n in range(_NH)], axis=1)
    hcat_bf = hcat.astype(_bf16)
    QK = _HEADS * _CK
    qcat = _dot(hcat_bf, cq_ref[...]).astype(_bf16)
    kcat = _dot(hcat_bf, ck_ref[...]).astype(_bf16)
    vcat = _dot(hcat_bf, cv_ref[...]).astype(_bf16)
    qrep = jnp.concatenate(
        [qcat[:, n * QK:(n + 1) * QK] for n in range(_NH) for _ in range(_NH)],
        axis=1)
    krep2 = jnp.concatenate([kcat] * _NH, axis=1)
    ls = _dot(qrep * krep2, S3_ref[...]) * (1.0 / math.sqrt(_CK))
    e4 = jnp.exp(ls)
    den = _dot(e4.astype(_bf16), Gden_ref[...])
    p = (e4 * (1.0 / den)).astype(_bf16)
    pexp = _dot(p, Expand_ref[...]).astype(_bf16)
    vtile = jnp.concatenate([vcat] * _NH, axis=1)
    ctx = _dot(pexp * vtile, Collapse_ref[...])
    out_ref[...] = hcat + ctx


def kernel(x, hs, rule_embeddings, pa_Wq, pa_bq, pa_Wk, pa_bk, pa_Wv, pa_bv,
           sa_Wq, sa_Wk, gru_Wih, gru_Whh, gru_bih, gru_bhh, c_Wq, c_Wk, c_Wv):
    B, IN = x.shape
    NH, Hd = hs.shape[1], hs.shape[2]
    Bt = 1024
    G = B // Bt
    hs_f = hs.reshape(B * NH, Hd)
    bf = lambda a: a.astype(_bf16)
    Wih_all = bf(gru_Wih.transpose(0, 2, 1)).reshape(_R * gru_Wih.shape[2], -1)
    Whh_all = bf(gru_Whh.transpose(0, 2, 1)).reshape(_R * gru_Whh.shape[2], -1)
    eye = jnp.eye(NH, dtype=jnp.float32)
    cq_blk = bf(jnp.kron(eye, c_Wq))
    ck_blk = bf(jnp.kron(eye, c_Wk))
    cv_blk = bf(jnp.kron(eye, c_Wv))

    QK = _HEADS * _CK
    l_i = jnp.arange(NH * NH * QK)[:, None]
    j_c = jnp.arange(NH * _HEADS * NH)[None, :]
    S3 = bf((l_i // (NH * QK) == j_c // (_HEADS * NH))
            & ((l_i // QK) % NH == j_c % NH)
            & ((l_i % QK) // _CK == (j_c // NH) % _HEADS))
    j2 = jnp.arange(NH * _HEADS * NH)
    Gden = bf(j2[:, None] // NH == j2[None, :] // NH)
    j_r = jnp.arange(NH * _HEADS * NH)[:, None]
    e_c = jnp.arange(NH * NH * _HEADS * _VH)[None, :]
    Expand = bf((j_r // (_HEADS * NH) == e_c // (NH * Hd))
                & (j_r % NH == (e_c // (_HEADS * _VH)) % NH)
                & ((j_r // NH) % _HEADS == (e_c // _VH) % _HEADS))
    e_r = jnp.arange(NH * NH * _HEADS * _VH)[:, None]
    o_c = jnp.arange(NH * Hd)[None, :]
    Collapse = bf((e_r // (NH * Hd) == o_c // Hd)
                  & (e_r % (_HEADS * _VH) == o_c % Hd))

    def row2(v):
        return v.reshape(1, -1)

    full2 = lambda a: pl.BlockSpec(a.shape, lambda i: (0, 0))
    args = (x, hs_f, rule_embeddings, bf(pa_Wq), row2(pa_bq), bf(pa_Wk),
            row2(pa_bk), bf(pa_Wv), row2(pa_bv), bf(sa_Wq), bf(sa_Wk),
            Wih_all, Whh_all, bf(gru_bih), bf(gru_bhh), cq_blk, ck_blk,
            cv_blk, S3, Gden, Expand, Collapse)
    out = pl.pallas_call(
        _fused_kernel,
        grid=(G,),
        in_specs=[
            pl.BlockSpec((Bt, IN), lambda i: (i, 0)),
            pl.BlockSpec((Bt * NH, Hd), lambda i: (i, 0)),
        ] + [full2(a) for a in args[2:]],
        out_specs=pl.BlockSpec((Bt, NH * Hd), lambda i: (i, 0)),
        out_shape=jax.ShapeDtypeStruct((B, NH * Hd), jnp.float32),
    )(*args)
    return out.reshape(B, NH, Hd)
